# baseline (speedup 1.0000x reference)
import functools

import jax
import jax.numpy as jnp
from jax import lax
from jax.experimental import pallas as pl
from jax.experimental.pallas import tpu as pltpu
from jax.experimental.pallas import tpu_sc as plsc

BATCH = 8
SEQ = 1024
RED = 1024
DMODEL = 512
NROWS = BATCH * SEQ

NUM_WORKERS = 32
ROWS_PER_WORKER = NROWS // NUM_WORKERS
CHUNK = 128


def _counts_body(d_ref, idx_ref):
    d = d_ref[...]
    hit = jnp.logical_or(d == 1, d == -1)
    c = jnp.sum(hit.astype(jnp.int32), axis=-1)
    idx_ref[...] = jnp.minimum(c, DMODEL - 1).reshape(1, 1, SEQ)


def _centrality_counts(distances):
    return pl.pallas_call(
        _counts_body,
        grid=(BATCH,),
        in_specs=[pl.BlockSpec((1, SEQ, RED), lambda b: (b, 0, 0))],
        out_specs=pl.BlockSpec((1, 1, SEQ), lambda b: (b, 0, 0)),
        out_shape=jax.ShapeDtypeStruct((BATCH, 1, SEQ), jnp.int32),
    )(distances)


def _gather_body(table_hbm, idx_hbm, out_hbm, idx_v, rows_v, sem):
    wid = lax.axis_index("s") * 2 + lax.axis_index("c")
    base = wid * ROWS_PER_WORKER
    pltpu.sync_copy(idx_hbm.at[pl.ds(base, ROWS_PER_WORKER)], idx_v)
    for c in range(ROWS_PER_WORKER // CHUNK):
        pltpu.async_copy(
            table_hbm.at[idx_v.at[pl.ds(c * CHUNK, CHUNK)]], rows_v, sem
        ).wait()
        pltpu.sync_copy(rows_v, out_hbm.at[pl.ds(base + c * CHUNK, CHUNK)])


@functools.lru_cache(maxsize=1)
def _gather_rows():
    return pl.kernel(
        _gather_body,
        mesh=plsc.VectorSubcoreMesh(core_axis_name="c", subcore_axis_name="s"),
        out_type=jax.ShapeDtypeStruct((NROWS, DMODEL), jnp.float32),
        scratch_types=[
            pltpu.VMEM((ROWS_PER_WORKER,), jnp.int32),
            pltpu.VMEM((CHUNK, DMODEL), jnp.float32),
            pltpu.SemaphoreType.DMA,
        ],
    )


def kernel(distances, table):
    idx = _centrality_counts(distances).reshape(NROWS)
    rows = _gather_rows()(table, idx)
    return rows.reshape(BATCH, SEQ, DMODEL)
